# baseline (speedup 1.0000x reference)
import functools

import jax
import jax.numpy as jnp
from jax import lax
from jax.experimental import pallas as pl
from jax.experimental.pallas import tpu as pltpu
from jax.experimental.pallas import tpu_sc as plsc

N = 10000
E = 320000
D = 128
NC = 2
NS = 16
EPW = E // (NC * NS)
C = 50
NCHUNK = EPW // C
RPT = N // NS
RZ = 125
RCH = RPT // RZ

_mesh = plsc.VectorSubcoreMesh(
    core_axis_name="c", subcore_axis_name="s", num_cores=NC, num_subcores=NS)


def _make_pool():
    def body(h_hbm, src_hbm, dst_hbm, zer_hbm, zed_hbm, one_hbm,
             part_out, deg_out,
             src_v, dst_v, r0, r1, ones_v,
             g0, g1, a0, a1, d0, d1,
             agg_sh, deg_sh):
        rows = (r0, r1)
        gsem = (g0, g1)
        asem = (a0, a1)
        dsem = (d0, d1)
        c = lax.axis_index("c")
        s = lax.axis_index("s")

        def zbody(i, carry):
            base = s * RPT + i * RZ
            pltpu.sync_copy(zer_hbm, agg_sh.at[pl.ds(base, RZ)])
            pltpu.sync_copy(zed_hbm, deg_sh.at[pl.ds(base, RZ)])
            return carry
        lax.fori_loop(0, RCH, zbody, 0)

        pltpu.sync_copy(src_hbm.at[c, s], src_v)
        pltpu.sync_copy(dst_hbm.at[c, s], dst_v)
        pltpu.sync_copy(one_hbm, ones_v)
        plsc.subcore_barrier()

        def wait_adds(j, b):
            pltpu.make_async_copy(rows[b], agg_sh.at[dst_v.at[j]],
                                  asem[b]).wait()
            pltpu.make_async_copy(ones_v, deg_sh.at[dst_v.at[j]],
                                  dsem[b]).wait()

        def issue_gather(j, b):
            pltpu.async_copy(h_hbm.at[src_v.at[j]], rows[b], gsem[b])

        def wait_gather(j, b):
            pltpu.make_async_copy(h_hbm.at[src_v.at[j]], rows[b],
                                  gsem[b]).wait()

        def issue_adds(j, b):
            pltpu.async_copy(rows[b], agg_sh.at[dst_v.at[j]], asem[b],
                             add=True)
            pltpu.async_copy(ones_v, deg_sh.at[dst_v.at[j]], dsem[b],
                             add=True)

        issue_gather(0, 0)

        def outer(i0, carry):
            for half in range(2):
                j = i0 * 2 + half
                b = half
                ob = 1 - half
                wait_gather(j, b)
                issue_adds(j, b)
                if half == 1:
                    wait_adds(j - 1, ob)

                    @pl.when(i0 < NCHUNK // 2 - 1)
                    def _():
                        issue_gather(j + 1, ob)
                else:
                    @pl.when(i0 >= 1)
                    def _():
                        wait_adds(j - 1, ob)
                    issue_gather(j + 1, ob)
            return carry
        lax.fori_loop(0, NCHUNK // 2, outer, 0)

        wait_adds(NCHUNK - 1, 1)
        plsc.subcore_barrier()

        def obody(i, carry):
            base = s * RPT + i * RZ
            pltpu.sync_copy(agg_sh.at[pl.ds(base, RZ)],
                            part_out.at[c, pl.ds(base, RZ)])
            pltpu.sync_copy(deg_sh.at[pl.ds(base, RZ)],
                            deg_out.at[c, pl.ds(base, RZ)])
            return carry
        lax.fori_loop(0, RCH, obody, 0)

    return pl.kernel(
        body,
        out_type=(jax.ShapeDtypeStruct((NC, N, D), jnp.float32),
                  jax.ShapeDtypeStruct((NC, N, 16), jnp.float32)),
        mesh=_mesh,
        compiler_params=pltpu.CompilerParams(use_tc_tiling_on_sc=False),
        scratch_types=(
            pltpu.VMEM((NCHUNK, C), jnp.int32),
            pltpu.VMEM((NCHUNK, C), jnp.int32),
            pltpu.VMEM((C, D), jnp.float32),
            pltpu.VMEM((C, D), jnp.float32),
            pltpu.VMEM((C, 16), jnp.float32),
        ) + (pltpu.SemaphoreType.DMA,) * 6 + (
            pltpu.VMEM_SHARED((N, D), jnp.float32),
            pltpu.VMEM_SHARED((N, 16), jnp.float32),
        ),
    )


_pool = _make_pool()


RB = 1000


def _gemm1_body(x_ref, w_ref, b_ref, o_ref):
    o_ref[...] = (jnp.dot(x_ref[...], w_ref[...],
                          preferred_element_type=jnp.float32)
                  + b_ref[...][None, :])


_gemm1 = pl.pallas_call(
    _gemm1_body,
    grid=(N // RB,),
    in_specs=[
        pl.BlockSpec((RB, D), lambda i: (i, 0)),
        pl.BlockSpec((D, D), lambda i: (0, 0)),
        pl.BlockSpec((D,), lambda i: (0,)),
    ],
    out_specs=pl.BlockSpec((RB, D), lambda i: (i, 0)),
    out_shape=jax.ShapeDtypeStruct((N, D), jnp.float32),
)


def _norm(p_ref, d_ref):
    p = p_ref[0] + p_ref[1]
    deg = jnp.sum(d_ref[0] + d_ref[1], axis=1, keepdims=True) / 16.0
    return p / jnp.maximum(deg, 1.0)


def _comb_gemm_body(p_ref, d_ref, w_ref, b_ref, o_ref):
    pooled = _norm(p_ref, d_ref)
    o_ref[...] = (jnp.dot(pooled, w_ref[...],
                          preferred_element_type=jnp.float32)
                  + b_ref[...][None, :])


_comb_gemm = pl.pallas_call(
    _comb_gemm_body,
    grid=(N // RB,),
    in_specs=[
        pl.BlockSpec((NC, RB, D), lambda i: (0, i, 0)),
        pl.BlockSpec((NC, RB, 16), lambda i: (0, i, 0)),
        pl.BlockSpec((D, D), lambda i: (0, 0)),
        pl.BlockSpec((D,), lambda i: (0,)),
    ],
    out_specs=pl.BlockSpec((RB, D), lambda i: (i, 0)),
    out_shape=jax.ShapeDtypeStruct((N, D), jnp.float32),
)


def _comb_body(p_ref, d_ref, o_ref):
    o_ref[...] = _norm(p_ref, d_ref)


_comb = pl.pallas_call(
    _comb_body,
    grid=(N // RB,),
    in_specs=[
        pl.BlockSpec((NC, RB, D), lambda i: (0, i, 0)),
        pl.BlockSpec((NC, RB, 16), lambda i: (0, i, 0)),
    ],
    out_specs=pl.BlockSpec((RB, D), lambda i: (i, 0)),
    out_shape=jax.ShapeDtypeStruct((N, D), jnp.float32),
)


def kernel(x, edge_index, W1, b1, W2, b2):
    src = edge_index[0].reshape(NC, NS, NCHUNK, C)
    dst = edge_index[1].reshape(NC, NS, NCHUNK, C)
    zer = jnp.zeros((RZ, D), jnp.float32)
    zed = jnp.zeros((RZ, 16), jnp.float32)
    one = jnp.ones((C, 16), jnp.float32)

    h1 = _gemm1(x, W1, b1)
    p1, d1 = _pool(h1, src, dst, zer, zed, one)
    h2 = _comb_gemm(p1, d1, W2, b2)
    p2, _ = _pool(h2, src, dst, zer, zed, one)
    return _comb(p2, d1)


# speedup vs baseline: 1.4757x; 1.4757x over previous
import functools

import jax
import jax.numpy as jnp
from jax import lax
from jax.experimental import pallas as pl
from jax.experimental.pallas import tpu as pltpu
from jax.experimental.pallas import tpu_sc as plsc

N = 10000
E = 320000
D = 128
NC = 2
NS = 16
EPW = E // (NC * NS)
C = 125
NCHUNK = EPW // C
W = 8
NG = NCHUNK // W
RPT = N // NS
RZ = 125
RCH = RPT // RZ

_mesh = plsc.VectorSubcoreMesh(
    core_axis_name="c", subcore_axis_name="s", num_cores=NC, num_subcores=NS)


def _make_pool():
    def body(h_hbm, src_hbm, dst_hbm, zer_hbm, zed_hbm, one_hbm,
             part_out, deg_out,
             src_w, dst_w, r0, r1, ones_v,
             g0, g1, a0, a1, d0, d1, ws0, ws1, wd0, wd1,
             agg_sh, deg_sh):
        rows = (r0, r1)
        gsem = (g0, g1)
        asem = (a0, a1)
        dsem = (d0, d1)
        wssem = (ws0, ws1)
        wdsem = (wd0, wd1)
        c = lax.axis_index("c")
        s = lax.axis_index("s")

        def zbody(i, carry):
            base = s * RPT + i * RZ
            pltpu.sync_copy(zer_hbm, agg_sh.at[pl.ds(base, RZ)])
            pltpu.sync_copy(zed_hbm, deg_sh.at[pl.ds(base, RZ)])
            return carry
        lax.fori_loop(0, RCH, zbody, 0)

        def issue_window(g, slot):
            pltpu.async_copy(src_hbm.at[c, s, pl.ds(g * W, W)],
                             src_w.at[slot], wssem[slot])
            pltpu.async_copy(dst_hbm.at[c, s, pl.ds(g * W, W)],
                             dst_w.at[slot], wdsem[slot])

        def wait_window(slot):
            pltpu.make_async_copy(src_hbm.at[c, s, pl.ds(0, W)],
                                  src_w.at[slot], wssem[slot]).wait()
            pltpu.make_async_copy(dst_hbm.at[c, s, pl.ds(0, W)],
                                  dst_w.at[slot], wdsem[slot]).wait()

        def wait_adds(b):
            pltpu.make_async_copy(rows[b], agg_sh.at[dst_w.at[0, 0]],
                                  asem[b]).wait()
            pltpu.make_async_copy(ones_v, deg_sh.at[dst_w.at[0, 0]],
                                  dsem[b]).wait()

        def issue_gather(slot, wb, b):
            pltpu.async_copy(h_hbm.at[src_w.at[slot, wb]], rows[b], gsem[b])

        def wait_gather(b):
            pltpu.make_async_copy(h_hbm.at[src_w.at[0, 0]], rows[b],
                                  gsem[b]).wait()

        def issue_adds(slot, wb, b):
            pltpu.async_copy(rows[b], agg_sh.at[dst_w.at[slot, wb]], asem[b],
                             add=True)
            pltpu.async_copy(ones_v, deg_sh.at[dst_w.at[slot, wb]], dsem[b],
                             add=True)

        pltpu.sync_copy(one_hbm, ones_v)
        issue_window(0, 0)
        wait_window(0)
        plsc.subcore_barrier()
        issue_gather(0, 0, 0)

        def outer(i0, carry):
            for k in range(2 * W):
                slot = k // W
                wb = k % W
                b = k % 2
                ob = 1 - b
                nslot = 1 - slot
                wait_gather(b)
                issue_adds(slot, wb, b)
                if wb == 2:
                    if slot == 0:
                        issue_window(2 * i0 + 1, 1)
                    else:
                        @pl.when(i0 < NG // 2 - 1)
                        def _():
                            issue_window(2 * i0 + 2, 0)
                if k == 0:
                    @pl.when(i0 >= 1)
                    def _():
                        wait_adds(ob)
                    issue_gather(slot, wb + 1, ob)
                elif wb == W - 1:
                    wait_adds(ob)
                    if slot == 0:
                        wait_window(1)
                        issue_gather(1, 0, ob)
                    else:
                        @pl.when(i0 < NG // 2 - 1)
                        def _():
                            wait_window(0)
                            issue_gather(0, 0, ob)
                else:
                    wait_adds(ob)
                    issue_gather(slot, wb + 1, ob)
            return carry
        lax.fori_loop(0, NG // 2, outer, 0)

        wait_adds(1)
        plsc.subcore_barrier()

        def obody(i, carry):
            base = s * RPT + i * RZ
            pltpu.sync_copy(agg_sh.at[pl.ds(base, RZ)],
                            part_out.at[c, pl.ds(base, RZ)])
            pltpu.sync_copy(deg_sh.at[pl.ds(base, RZ)],
                            deg_out.at[c, pl.ds(base, RZ)])
            return carry
        lax.fori_loop(0, RCH, obody, 0)

    return pl.kernel(
        body,
        out_type=(jax.ShapeDtypeStruct((NC, N, D), jnp.float32),
                  jax.ShapeDtypeStruct((NC, N, 16), jnp.float32)),
        mesh=_mesh,
        compiler_params=pltpu.CompilerParams(use_tc_tiling_on_sc=False),
        scratch_types=(
            pltpu.VMEM((2, W, C), jnp.int32),
            pltpu.VMEM((2, W, C), jnp.int32),
            pltpu.VMEM((C, D), jnp.float32),
            pltpu.VMEM((C, D), jnp.float32),
            pltpu.VMEM((C, 16), jnp.float32),
        ) + (pltpu.SemaphoreType.DMA,) * 10 + (
            pltpu.VMEM_SHARED((N, D), jnp.float32),
            pltpu.VMEM_SHARED((N, 16), jnp.float32),
        ),
    )


_pool = _make_pool()


RB = 1000


def _gemm1_body(x_ref, w_ref, b_ref, o_ref):
    o_ref[...] = (jnp.dot(x_ref[...], w_ref[...],
                          preferred_element_type=jnp.float32)
                  + b_ref[...][None, :])


_gemm1 = pl.pallas_call(
    _gemm1_body,
    grid=(N // RB,),
    in_specs=[
        pl.BlockSpec((RB, D), lambda i: (i, 0)),
        pl.BlockSpec((D, D), lambda i: (0, 0)),
        pl.BlockSpec((D,), lambda i: (0,)),
    ],
    out_specs=pl.BlockSpec((RB, D), lambda i: (i, 0)),
    out_shape=jax.ShapeDtypeStruct((N, D), jnp.float32),
)


def _norm(p_ref, d_ref):
    p = p_ref[0] + p_ref[1]
    deg = jnp.sum(d_ref[0] + d_ref[1], axis=1, keepdims=True) / 16.0
    return p / jnp.maximum(deg, 1.0)


def _comb_gemm_body(p_ref, d_ref, w_ref, b_ref, o_ref):
    pooled = _norm(p_ref, d_ref)
    o_ref[...] = (jnp.dot(pooled, w_ref[...],
                          preferred_element_type=jnp.float32)
                  + b_ref[...][None, :])


_comb_gemm = pl.pallas_call(
    _comb_gemm_body,
    grid=(N // RB,),
    in_specs=[
        pl.BlockSpec((NC, RB, D), lambda i: (0, i, 0)),
        pl.BlockSpec((NC, RB, 16), lambda i: (0, i, 0)),
        pl.BlockSpec((D, D), lambda i: (0, 0)),
        pl.BlockSpec((D,), lambda i: (0,)),
    ],
    out_specs=pl.BlockSpec((RB, D), lambda i: (i, 0)),
    out_shape=jax.ShapeDtypeStruct((N, D), jnp.float32),
)


def _comb_body(p_ref, d_ref, o_ref):
    o_ref[...] = _norm(p_ref, d_ref)


_comb = pl.pallas_call(
    _comb_body,
    grid=(N // RB,),
    in_specs=[
        pl.BlockSpec((NC, RB, D), lambda i: (0, i, 0)),
        pl.BlockSpec((NC, RB, 16), lambda i: (0, i, 0)),
    ],
    out_specs=pl.BlockSpec((RB, D), lambda i: (i, 0)),
    out_shape=jax.ShapeDtypeStruct((N, D), jnp.float32),
)


def kernel(x, edge_index, W1, b1, W2, b2):
    src = edge_index[0].reshape(NC, NS, NCHUNK, C)
    dst = edge_index[1].reshape(NC, NS, NCHUNK, C)
    zer = jnp.zeros((RZ, D), jnp.float32)
    zed = jnp.zeros((RZ, 16), jnp.float32)
    one = jnp.ones((C, 16), jnp.float32)

    h1 = _gemm1(x, W1, b1)
    p1, d1 = _pool(h1, src, dst, zer, zed, one)
    h2 = _comb_gemm(p1, d1, W2, b2)
    p2, _ = _pool(h2, src, dst, zer, zed, one)
    return _comb(p2, d1)


# speedup vs baseline: 1.4792x; 1.0024x over previous
import functools

import jax
import jax.numpy as jnp
from jax import lax
from jax.experimental import pallas as pl
from jax.experimental.pallas import tpu as pltpu
from jax.experimental.pallas import tpu_sc as plsc

N = 10000
E = 320000
D = 128
NC = 2
NS = 16
EPW = E // (NC * NS)
C = 125
NCHUNK = EPW // C
W = 8
NG = NCHUNK // W
RPT = N // NS
RZ = 125
RCH = RPT // RZ

_mesh = plsc.VectorSubcoreMesh(
    core_axis_name="c", subcore_axis_name="s", num_cores=NC, num_subcores=NS)


def _make_pool():
    def body(h_hbm, src_hbm, dst_hbm, zer_hbm, zed_hbm, one_hbm, flg_hbm,
             part_out, deg_out,
             src_w, dst_w, r0, r1, ones_v, flg_v,
             g0, g1, a0, a1, d0, d1, ws0, ws1, wd0, wd1,
             agg_sh, deg_sh):
        rows = (r0, r1)
        gsem = (g0, g1)
        asem = (a0, a1)
        dsem = (d0, d1)
        wssem = (ws0, ws1)
        wdsem = (wd0, wd1)
        c = lax.axis_index("c")
        s = lax.axis_index("s")

        pltpu.sync_copy(flg_hbm, flg_v)
        with_deg = jnp.sum(flg_v[...]) > 0

        def zbody(i, carry):
            base = s * RPT + i * RZ
            pltpu.sync_copy(zer_hbm, agg_sh.at[pl.ds(base, RZ)])

            @pl.when(with_deg)
            def _():
                pltpu.sync_copy(zed_hbm, deg_sh.at[pl.ds(base, RZ)])
            return carry
        lax.fori_loop(0, RCH, zbody, 0)

        def issue_window(g, slot):
            pltpu.async_copy(src_hbm.at[c, s, pl.ds(g * W, W)],
                             src_w.at[slot], wssem[slot])
            pltpu.async_copy(dst_hbm.at[c, s, pl.ds(g * W, W)],
                             dst_w.at[slot], wdsem[slot])

        def wait_window(slot):
            pltpu.make_async_copy(src_hbm.at[c, s, pl.ds(0, W)],
                                  src_w.at[slot], wssem[slot]).wait()
            pltpu.make_async_copy(dst_hbm.at[c, s, pl.ds(0, W)],
                                  dst_w.at[slot], wdsem[slot]).wait()

        def wait_adds(b):
            pltpu.make_async_copy(rows[b], agg_sh.at[dst_w.at[0, 0]],
                                  asem[b]).wait()

            @pl.when(with_deg)
            def _():
                pltpu.make_async_copy(ones_v, deg_sh.at[dst_w.at[0, 0]],
                                      dsem[b]).wait()

        def issue_gather(slot, wb, b):
            pltpu.async_copy(h_hbm.at[src_w.at[slot, wb]], rows[b], gsem[b])

        def wait_gather(b):
            pltpu.make_async_copy(h_hbm.at[src_w.at[0, 0]], rows[b],
                                  gsem[b]).wait()

        def issue_adds(slot, wb, b):
            pltpu.async_copy(rows[b], agg_sh.at[dst_w.at[slot, wb]], asem[b],
                             add=True)

            @pl.when(with_deg)
            def _():
                pltpu.async_copy(ones_v, deg_sh.at[dst_w.at[slot, wb]],
                                 dsem[b], add=True)

        pltpu.sync_copy(one_hbm, ones_v)
        issue_window(0, 0)
        wait_window(0)
        plsc.subcore_barrier()
        issue_gather(0, 0, 0)

        def outer(i0, carry):
            for k in range(2 * W):
                slot = k // W
                wb = k % W
                b = k % 2
                ob = 1 - b
                nslot = 1 - slot
                wait_gather(b)
                issue_adds(slot, wb, b)
                if wb == 2:
                    if slot == 0:
                        issue_window(2 * i0 + 1, 1)
                    else:
                        @pl.when(i0 < NG // 2 - 1)
                        def _():
                            issue_window(2 * i0 + 2, 0)
                if k == 0:
                    @pl.when(i0 >= 1)
                    def _():
                        wait_adds(ob)
                    issue_gather(slot, wb + 1, ob)
                elif wb == W - 1:
                    wait_adds(ob)
                    if slot == 0:
                        wait_window(1)
                        issue_gather(1, 0, ob)
                    else:
                        @pl.when(i0 < NG // 2 - 1)
                        def _():
                            wait_window(0)
                            issue_gather(0, 0, ob)
                else:
                    wait_adds(ob)
                    issue_gather(slot, wb + 1, ob)
            return carry
        lax.fori_loop(0, NG // 2, outer, 0)

        wait_adds(1)
        plsc.subcore_barrier()

        def obody(i, carry):
            base = s * RPT + i * RZ
            pltpu.sync_copy(agg_sh.at[pl.ds(base, RZ)],
                            part_out.at[c, pl.ds(base, RZ)])

            @pl.when(with_deg)
            def _():
                pltpu.sync_copy(deg_sh.at[pl.ds(base, RZ)],
                                deg_out.at[c, pl.ds(base, RZ)])
            return carry
        lax.fori_loop(0, RCH, obody, 0)

    return pl.kernel(
        body,
        out_type=(jax.ShapeDtypeStruct((NC, N, D), jnp.float32),
                  jax.ShapeDtypeStruct((NC, N, 16), jnp.float32)),
        mesh=_mesh,
        compiler_params=pltpu.CompilerParams(use_tc_tiling_on_sc=False,
                                             needs_layout_passes=False),
        scratch_types=(
            pltpu.VMEM((2, W, C), jnp.int32),
            pltpu.VMEM((2, W, C), jnp.int32),
            pltpu.VMEM((C, D), jnp.float32),
            pltpu.VMEM((C, D), jnp.float32),
            pltpu.VMEM((C, 16), jnp.float32),
            pltpu.VMEM((16,), jnp.int32),
        ) + (pltpu.SemaphoreType.DMA,) * 10 + (
            pltpu.VMEM_SHARED((N, D), jnp.float32),
            pltpu.VMEM_SHARED((N, 16), jnp.float32),
        ),
    )


_pool = _make_pool()


RB = 1000


def _gemm1_body(x_ref, w_ref, b_ref, o_ref):
    o_ref[...] = (jnp.dot(x_ref[...], w_ref[...],
                          preferred_element_type=jnp.float32)
                  + b_ref[...][None, :])


_gemm1 = pl.pallas_call(
    _gemm1_body,
    grid=(N // RB,),
    in_specs=[
        pl.BlockSpec((RB, D), lambda i: (i, 0)),
        pl.BlockSpec((D, D), lambda i: (0, 0)),
        pl.BlockSpec((D,), lambda i: (0,)),
    ],
    out_specs=pl.BlockSpec((RB, D), lambda i: (i, 0)),
    out_shape=jax.ShapeDtypeStruct((N, D), jnp.float32),
)


def _norm(p_ref, d_ref):
    p = p_ref[0] + p_ref[1]
    deg = jnp.sum(d_ref[0] + d_ref[1], axis=1, keepdims=True) / 16.0
    return p / jnp.maximum(deg, 1.0)


def _comb_gemm_body(p_ref, d_ref, w_ref, b_ref, o_ref):
    pooled = _norm(p_ref, d_ref)
    o_ref[...] = (jnp.dot(pooled, w_ref[...],
                          preferred_element_type=jnp.float32)
                  + b_ref[...][None, :])


_comb_gemm = pl.pallas_call(
    _comb_gemm_body,
    grid=(N // RB,),
    in_specs=[
        pl.BlockSpec((NC, RB, D), lambda i: (0, i, 0)),
        pl.BlockSpec((NC, RB, 16), lambda i: (0, i, 0)),
        pl.BlockSpec((D, D), lambda i: (0, 0)),
        pl.BlockSpec((D,), lambda i: (0,)),
    ],
    out_specs=pl.BlockSpec((RB, D), lambda i: (i, 0)),
    out_shape=jax.ShapeDtypeStruct((N, D), jnp.float32),
)


def _comb_body(p_ref, d_ref, o_ref):
    o_ref[...] = _norm(p_ref, d_ref)


_comb = pl.pallas_call(
    _comb_body,
    grid=(N // RB,),
    in_specs=[
        pl.BlockSpec((NC, RB, D), lambda i: (0, i, 0)),
        pl.BlockSpec((NC, RB, 16), lambda i: (0, i, 0)),
    ],
    out_specs=pl.BlockSpec((RB, D), lambda i: (i, 0)),
    out_shape=jax.ShapeDtypeStruct((N, D), jnp.float32),
)


def kernel(x, edge_index, W1, b1, W2, b2):
    src = edge_index[0].reshape(NC, NS, NCHUNK, C)
    dst = edge_index[1].reshape(NC, NS, NCHUNK, C)
    zer = jnp.zeros((RZ, D), jnp.float32)
    zed = jnp.zeros((RZ, 16), jnp.float32)
    one = jnp.ones((C, 16), jnp.float32)

    f1 = jnp.ones((16,), jnp.int32)
    f0 = jnp.zeros((16,), jnp.int32)

    h1 = _gemm1(x, W1, b1)
    p1, d1 = _pool(h1, src, dst, zer, zed, one, f1)
    h2 = _comb_gemm(p1, d1, W2, b2)
    p2, _ = _pool(h2, src, dst, zer, zed, one, f0)
    return _comb(p2, d1)


# speedup vs baseline: 1.5097x; 1.0206x over previous
import functools

import jax
import jax.numpy as jnp
from jax import lax
from jax.experimental import pallas as pl
from jax.experimental.pallas import tpu as pltpu
from jax.experimental.pallas import tpu_sc as plsc

N = 10000
E = 320000
D = 128
NC = 2
NS = 16
EPW = E // (NC * NS)
C = 125
NCHUNK = EPW // C
W = 8
NG = NCHUNK // W
RPT = N // NS
RZ = 125
RCH = RPT // RZ

_mesh = plsc.VectorSubcoreMesh(
    core_axis_name="c", subcore_axis_name="s", num_cores=NC, num_subcores=NS)


def _make_pool():
    def body(h_hbm, src_hbm, dst_hbm, zer_hbm, zed_hbm, one_hbm, flg_hbm,
             part_out, deg_out,
             src_w, dst_w, r0, r1, ones_v, flg_v,
             g0, g1, a0, a1, d0, d1, ws0, ws1, wd0, wd1,
             agg_sh, deg_sh):
        rows = (r0, r1)
        gsem = (g0, g1)
        asem = (a0, a1)
        dsem = (d0, d1)
        wssem = (ws0, ws1)
        wdsem = (wd0, wd1)
        c = lax.axis_index("c")
        s = lax.axis_index("s")

        pltpu.sync_copy(flg_hbm, flg_v)
        with_deg = jnp.sum(flg_v[...]) > 0

        def zbody(i, carry):
            base = s * RPT + i * RZ
            pltpu.sync_copy(zer_hbm, agg_sh.at[pl.ds(base, RZ)])

            @pl.when(with_deg)
            def _():
                pltpu.sync_copy(zed_hbm, deg_sh.at[pl.ds(base, RZ)])
            return carry
        lax.fori_loop(0, RCH, zbody, 0)

        def issue_window(g, slot):
            pltpu.async_copy(src_hbm.at[c, s, pl.ds(g * W, W)],
                             src_w.at[slot], wssem[slot])
            pltpu.async_copy(dst_hbm.at[c, s, pl.ds(g * W, W)],
                             dst_w.at[slot], wdsem[slot])

        def wait_window(slot):
            pltpu.make_async_copy(src_hbm.at[c, s, pl.ds(0, W)],
                                  src_w.at[slot], wssem[slot]).wait()
            pltpu.make_async_copy(dst_hbm.at[c, s, pl.ds(0, W)],
                                  dst_w.at[slot], wdsem[slot]).wait()

        def wait_adds(b):
            pltpu.make_async_copy(rows[b], agg_sh.at[dst_w.at[0, 0]],
                                  asem[b]).wait()

            @pl.when(with_deg)
            def _():
                pltpu.make_async_copy(ones_v, deg_sh.at[dst_w.at[0, 0]],
                                      dsem[b]).wait()

        def issue_gather(slot, wb, b):
            pltpu.async_copy(h_hbm.at[src_w.at[slot, wb]], rows[b], gsem[b])

        def wait_gather(b):
            pltpu.make_async_copy(h_hbm.at[src_w.at[0, 0]], rows[b],
                                  gsem[b]).wait()

        def issue_adds(slot, wb, b):
            pltpu.async_copy(rows[b], agg_sh.at[dst_w.at[slot, wb]], asem[b],
                             add=True)

            @pl.when(with_deg)
            def _():
                pltpu.async_copy(ones_v, deg_sh.at[dst_w.at[slot, wb]],
                                 dsem[b], add=True)

        pltpu.sync_copy(one_hbm, ones_v)
        issue_window(0, 0)
        wait_window(0)
        plsc.subcore_barrier()
        issue_gather(0, 0, 0)

        def outer(i0, carry):
            for k in range(2 * W):
                slot = k // W
                wb = k % W
                b = k % 2
                ob = 1 - b
                nslot = 1 - slot
                wait_gather(b)
                issue_adds(slot, wb, b)
                if wb == 2:
                    if slot == 0:
                        issue_window(2 * i0 + 1, 1)
                    else:
                        @pl.when(i0 < NG // 2 - 1)
                        def _():
                            issue_window(2 * i0 + 2, 0)
                if k == 0:
                    @pl.when(i0 >= 1)
                    def _():
                        wait_adds(ob)
                    issue_gather(slot, wb + 1, ob)
                elif wb == W - 1:
                    wait_adds(ob)
                    if slot == 0:
                        wait_window(1)
                        issue_gather(1, 0, ob)
                    else:
                        @pl.when(i0 < NG // 2 - 1)
                        def _():
                            wait_window(0)
                            issue_gather(0, 0, ob)
                else:
                    wait_adds(ob)
                    issue_gather(slot, wb + 1, ob)
            return carry
        lax.fori_loop(0, NG // 2, outer, 0)

        wait_adds(1)
        plsc.subcore_barrier()

        def obody(i, carry):
            base = s * RPT + i * RZ
            pltpu.sync_copy(agg_sh.at[pl.ds(base, RZ)],
                            part_out.at[c, pl.ds(base, RZ)])

            @pl.when(with_deg)
            def _():
                pltpu.sync_copy(deg_sh.at[pl.ds(base, RZ)],
                                deg_out.at[c, pl.ds(base, RZ)])
            return carry
        lax.fori_loop(0, RCH, obody, 0)

    return pl.kernel(
        body,
        out_type=(jax.ShapeDtypeStruct((NC, N, D), jnp.float32),
                  jax.ShapeDtypeStruct((NC, N, 16), jnp.float32)),
        mesh=_mesh,
        compiler_params=pltpu.CompilerParams(use_tc_tiling_on_sc=False,
                                             needs_layout_passes=False),
        scratch_types=(
            pltpu.VMEM((2, W, C), jnp.int32),
            pltpu.VMEM((2, W, C), jnp.int32),
            pltpu.VMEM((C, D), jnp.float32),
            pltpu.VMEM((C, D), jnp.float32),
            pltpu.VMEM((C, 16), jnp.float32),
            pltpu.VMEM((16,), jnp.int32),
        ) + (pltpu.SemaphoreType.DMA,) * 10 + (
            pltpu.VMEM_SHARED((N, D), jnp.float32),
            pltpu.VMEM_SHARED((N, 16), jnp.float32),
        ),
    )


_pool = _make_pool()


RB = 1000


def _wcomb_body(w1_ref, w2_ref, b1_ref, ow_ref, ob_ref):
    ow_ref[...] = jnp.dot(w1_ref[...], w2_ref[...],
                          preferred_element_type=jnp.float32)
    ob_ref[...] = jnp.dot(b1_ref[...][None, :], w2_ref[...],
                          preferred_element_type=jnp.float32)[0]


_wcomb = pl.pallas_call(
    _wcomb_body,
    out_shape=(jax.ShapeDtypeStruct((D, D), jnp.float32),
               jax.ShapeDtypeStruct((D,), jnp.float32)),
)


def _deg_of(d_ref):
    return jnp.sum(d_ref[0] + d_ref[1], axis=1, keepdims=True) / 16.0


def _norm(p_ref, d_ref):
    p = p_ref[0] + p_ref[1]
    return p / jnp.maximum(_deg_of(d_ref), 1.0)


def _comb_gemm_body(p_ref, d_ref, w_ref, b12_ref, b2_ref, o_ref):
    deg = _deg_of(d_ref)
    pooled = (p_ref[0] + p_ref[1]) / jnp.maximum(deg, 1.0)
    m = jnp.where(deg > 0.0, 1.0, 0.0)
    o_ref[...] = (jnp.dot(pooled, w_ref[...],
                          preferred_element_type=jnp.float32)
                  + m * b12_ref[...][None, :] + b2_ref[...][None, :])


_comb_gemm = pl.pallas_call(
    _comb_gemm_body,
    grid=(N // RB,),
    in_specs=[
        pl.BlockSpec((NC, RB, D), lambda i: (0, i, 0)),
        pl.BlockSpec((NC, RB, 16), lambda i: (0, i, 0)),
        pl.BlockSpec((D, D), lambda i: (0, 0)),
        pl.BlockSpec((D,), lambda i: (0,)),
        pl.BlockSpec((D,), lambda i: (0,)),
    ],
    out_specs=pl.BlockSpec((RB, D), lambda i: (i, 0)),
    out_shape=jax.ShapeDtypeStruct((N, D), jnp.float32),
)


def _comb_body(p_ref, d_ref, o_ref):
    o_ref[...] = _norm(p_ref, d_ref)


_comb = pl.pallas_call(
    _comb_body,
    grid=(N // RB,),
    in_specs=[
        pl.BlockSpec((NC, RB, D), lambda i: (0, i, 0)),
        pl.BlockSpec((NC, RB, 16), lambda i: (0, i, 0)),
    ],
    out_specs=pl.BlockSpec((RB, D), lambda i: (i, 0)),
    out_shape=jax.ShapeDtypeStruct((N, D), jnp.float32),
)


def kernel(x, edge_index, W1, b1, W2, b2):
    src = edge_index[0].reshape(NC, NS, NCHUNK, C)
    dst = edge_index[1].reshape(NC, NS, NCHUNK, C)
    zer = jnp.zeros((RZ, D), jnp.float32)
    zed = jnp.zeros((RZ, 16), jnp.float32)
    one = jnp.ones((C, 16), jnp.float32)

    f1 = jnp.ones((16,), jnp.int32)
    f0 = jnp.zeros((16,), jnp.int32)

    px, d1 = _pool(x, src, dst, zer, zed, one, f1)
    W12, b12 = _wcomb(W1, W2, b1)
    h2 = _comb_gemm(px, d1, W12, b12, b2)
    p2, _ = _pool(h2, src, dst, zer, zed, one, f0)
    return _comb(p2, d1)
